# baseline (speedup 1.0000x reference)
import jax
import jax.numpy as jnp
from jax.experimental import pallas as pl
from jax.experimental.pallas import tpu as pltpu

Q = 4096
K = 16384
D = 256
QB = 1024
KB = 8192
W = 256


def _top1_kernel(q_ref, k_ref, s_ref, i_ref):
    j = pl.program_id(1)
    q = q_ref[...]

    def chunk_scores(c):
        return jax.lax.dot_general(
            q, k_ref[c * W:(c + 1) * W, :],
            dimension_numbers=(((1,), (1,)), ((), ())),
            preferred_element_type=jnp.float32,
        )

    runm = chunk_scores(0)
    runc = jnp.zeros((QB, W), jnp.int32)
    for c in range(1, KB // W):
        v = chunk_scores(c)
        upd = v > runm
        runc = jnp.where(upd, c, runc)
        runm = jnp.maximum(runm, v)
    rowmax = jnp.max(runm, axis=1)
    lane = jax.lax.broadcasted_iota(jnp.int32, (QB, W), 1)
    cand = jnp.where(runm == rowmax[:, None], runc * W + lane, K)
    local = jnp.min(cand, axis=1) + j * KB

    @pl.when(j == 0)
    def _init():
        s_ref[...] = rowmax
        i_ref[...] = local

    @pl.when(j != 0)
    def _update():
        prev = s_ref[...]
        upd = rowmax > prev
        i_ref[...] = jnp.where(upd, local, i_ref[...])
        s_ref[...] = jnp.where(upd, rowmax, prev)


def kernel(queries, keys):
    grid = (Q // QB, K // KB)
    scores, idx = pl.pallas_call(
        _top1_kernel,
        grid=grid,
        in_specs=[
            pl.BlockSpec((QB, D), lambda i, j: (i, 0)),
            pl.BlockSpec((KB, D), lambda i, j: (j, 0)),
        ],
        out_specs=[
            pl.BlockSpec((QB,), lambda i, j: (i,)),
            pl.BlockSpec((QB,), lambda i, j: (i,)),
        ],
        out_shape=[
            jax.ShapeDtypeStruct((Q,), jnp.float32),
            jax.ShapeDtypeStruct((Q,), jnp.int32),
        ],
        compiler_params=pltpu.CompilerParams(
            dimension_semantics=("parallel", "arbitrary")),
    )(queries, keys)
    return scores, idx


# speedup vs baseline: 1.0155x; 1.0155x over previous
import jax
import jax.numpy as jnp
from jax.experimental import pallas as pl
from jax.experimental.pallas import tpu as pltpu

Q = 4096
K = 16384
D = 256
QB = 1024
KB = 16384
W = 256


def _top1_kernel(q_ref, k_ref, s_ref, i_ref):
    j = pl.program_id(1)
    q = q_ref[...]

    def chunk_scores(c):
        return jax.lax.dot_general(
            q, k_ref[c * W:(c + 1) * W, :],
            dimension_numbers=(((1,), (1,)), ((), ())),
            preferred_element_type=jnp.float32,
        )

    runm = chunk_scores(0)
    runc = jnp.zeros((QB, W), jnp.int32)
    for c in range(1, KB // W):
        v = chunk_scores(c)
        m = jnp.maximum(runm, v)
        upd = m > runm
        runc = jnp.where(upd, c, runc)
        runm = m
    rowmax = jnp.max(runm, axis=1)
    lane = jax.lax.broadcasted_iota(jnp.int32, (QB, W), 1)
    cand = jnp.where(runm == rowmax[:, None], runc * W + lane, K)
    local = jnp.min(cand, axis=1) + j * KB

    @pl.when(j == 0)
    def _init():
        s_ref[...] = rowmax
        i_ref[...] = local

    @pl.when(j != 0)
    def _update():
        prev = s_ref[...]
        upd = rowmax > prev
        i_ref[...] = jnp.where(upd, local, i_ref[...])
        s_ref[...] = jnp.where(upd, rowmax, prev)


def kernel(queries, keys):
    grid = (Q // QB, K // KB)
    scores, idx = pl.pallas_call(
        _top1_kernel,
        grid=grid,
        in_specs=[
            pl.BlockSpec((QB, D), lambda i, j: (i, 0)),
            pl.BlockSpec((KB, D), lambda i, j: (j, 0)),
        ],
        out_specs=[
            pl.BlockSpec((QB,), lambda i, j: (i,)),
            pl.BlockSpec((QB,), lambda i, j: (i,)),
        ],
        out_shape=[
            jax.ShapeDtypeStruct((Q,), jnp.float32),
            jax.ShapeDtypeStruct((Q,), jnp.int32),
        ],
        compiler_params=pltpu.CompilerParams(
            dimension_semantics=("parallel", "arbitrary")),
    )(queries, keys)
    return scores, idx
